# baseline (speedup 1.0000x reference)
import functools

import jax
import jax.numpy as jnp
from jax import lax
from jax.experimental import pallas as pl
from jax.experimental.pallas import tpu as pltpu
from jax.experimental.pallas import tpu_sc as plsc

R2 = 0.9 * 0.9
K = 32
N = 2048
S = 512
T = 4
B = 8
RT = T * B
MLP0 = 32
MLP1 = 32
NJOB = T * 3 * B
NROWS = NJOB * S * K
_NBK = 16
_BW = N // _NBK
_RND = 14
_CW = _NBK * _RND


def _fps_body(x_ref, y_ref, z_ref, ax_ref, ay_ref, az_ref):
    x = x_ref[...]
    y = y_ref[...]
    z = z_ref[...]
    iota_n = lax.broadcasted_iota(jnp.int32, (RT, N), 1)
    iota_s = lax.broadcasted_iota(jnp.int32, (RT, S), 1)

    def step(i, carry):
        dist, far, axc, ayc, azc = carry
        onehot = iota_n == far
        cx = jnp.sum(jnp.where(onehot, x, 0.0), axis=1, keepdims=True)
        cy = jnp.sum(jnp.where(onehot, y, 0.0), axis=1, keepdims=True)
        cz = jnp.sum(jnp.where(onehot, z, 0.0), axis=1, keepdims=True)
        axc = jnp.where(iota_s == i, cx, axc)
        ayc = jnp.where(iota_s == i, cy, ayc)
        azc = jnp.where(iota_s == i, cz, azc)
        dx = x - cx
        dy = y - cy
        dz = z - cz
        d = dx * dx + dy * dy + dz * dz
        dist = jnp.minimum(dist, d)
        m = jnp.max(dist, axis=1, keepdims=True)
        far = jnp.min(jnp.where(dist == m, iota_n, N), axis=1, keepdims=True)
        return dist, far, axc, ayc, azc

    dist0 = jnp.full((RT, N), 1e10, dtype=jnp.float32)
    far0 = jnp.zeros((RT, 1), dtype=jnp.int32)
    acc0 = jnp.zeros((RT, S), dtype=jnp.float32)
    _, _, axc, ayc, azc = lax.fori_loop(
        0, S, step, (dist0, far0, acc0, acc0, acc0))
    ax_ref[...] = axc
    ay_ref[...] = ayc
    az_ref[...] = azc


def _run_fps(xpl, ypl, zpl):
    out = jax.ShapeDtypeStruct((RT, S), jnp.float32)
    return pl.pallas_call(
        _fps_body,
        out_shape=(out, out, out),
    )(xpl, ypl, zpl)


def _ptab_body(x_ref, w_ref, o_ref):
    o_ref[...] = jnp.dot(x_ref[...], w_ref[...],
                         preferred_element_type=jnp.float32,
                         precision=lax.Precision.HIGHEST)


def _run_ptab(xyz8, wdt8):
    grid = 8
    rows = RT * N // grid
    return pl.pallas_call(
        _ptab_body,
        grid=(grid,),
        in_specs=[
            pl.BlockSpec((rows, 8), lambda g: (g, 0)),
            pl.BlockSpec((8, MLP0), lambda g: (0, 0)),
        ],
        out_specs=pl.BlockSpec((rows, MLP0), lambda g: (g, 0)),
        out_shape=jax.ShapeDtypeStruct((RT * N, MLP0), jnp.float32),
    )(xyz8, wdt8)


def _bq_body(axr, ayr, azr, xr, yr, zr, o_ref):
    t_id = pl.program_id(0)
    b_id = pl.program_id(1)
    dt_id = pl.program_id(2)
    i_frame = jnp.clip(t_id + dt_id - 1, 0, T - 1)
    base = (i_frame * B + b_id) * N

    ax = axr[...].reshape(S, 1)
    ay = ayr[...].reshape(S, 1)
    az = azr[...].reshape(S, 1)
    xn = xr[...].reshape(1, N)
    yn = yr[...].reshape(1, N)
    zn = zr[...].reshape(1, N)
    a8 = jnp.concatenate(
        [ax, ay, az, jnp.zeros((S, 5), jnp.float32)], axis=1)
    x8 = jnp.concatenate(
        [xn, yn, zn, jnp.zeros((5, N), jnp.float32)], axis=0)
    dot = jnp.dot(a8, x8, preferred_element_type=jnp.float32)
    sa = ax * ax + ay * ay + az * az
    sx = xn * xn + yn * yn + zn * zn
    dmat = (-2.0 * dot + sa) + sx

    iota_n = lax.broadcasted_iota(jnp.int32, (S, N), 1)
    iota_k = lax.broadcasted_iota(jnp.int32, (S, K), 1)
    iota_bw = lax.broadcasted_iota(jnp.int32, (S, _BW), 1)
    lane_cw = lax.broadcasted_iota(jnp.int32, (S, _CW), 1)

    def round_step(r, carry):
        dc, candd, candi = carry
        bms, sels, newd = [], [], []
        for bb in range(_NBK):
            sl = dc[:, bb * _BW:(bb + 1) * _BW]
            bm = jnp.min(sl, axis=1, keepdims=True)
            sb = jnp.min(jnp.where(sl == bm, iota_bw, _BW),
                         axis=1, keepdims=True)
            bms.append(bm)
            sels.append(sb + bb * _BW)
            newd.append(jnp.where(iota_bw == sb, jnp.inf, sl))
        dc = jnp.concatenate(newd, axis=1)
        cd = jnp.concatenate(bms, axis=1)
        ci = jnp.concatenate(sels, axis=1)
        sel_r = lane_cw // _NBK == r
        candd = jnp.where(sel_r, jnp.tile(cd, (1, _RND)), candd)
        candi = jnp.where(sel_r, jnp.tile(ci, (1, _RND)), candi)
        return dc, candd, candi

    candd0 = jnp.full((S, _CW), jnp.inf, dtype=jnp.float32)
    candi0 = jnp.full((S, _CW), N, dtype=jnp.int32)
    dc, candd, candi = lax.fori_loop(
        0, _RND, round_step, (dmat, candd0, candi0))

    minrem = jnp.min(dc, axis=1, keepdims=True)
    cnt = jnp.sum((candd < minrem).astype(jnp.int32), axis=1)
    safe = jnp.min(cnt) >= K

    def _extract(ed, ei):
        def step(k, carry):
            edc, idxs, sel0 = carry
            m = jnp.min(edc, axis=1, keepdims=True)
            sel = jnp.min(jnp.where(edc == m, ei, N), axis=1, keepdims=True)
            sel0 = jnp.where(k == 0, sel, sel0)
            eff = jnp.where(m > R2, sel0, sel)
            idxs = jnp.where(iota_k == k, eff, idxs)
            edc = jnp.where((edc == m) & (ei == sel), jnp.inf, edc)
            return edc, idxs, sel0

        idxs0 = jnp.zeros((S, K), dtype=jnp.int32)
        sel00 = jnp.zeros((S, 1), dtype=jnp.int32)
        _, idxs, _ = lax.fori_loop(0, K, step, (ed, idxs0, sel00))
        return idxs

    idxs = lax.cond(
        safe,
        lambda: _extract(candd, candi),
        lambda: _extract(jnp.concatenate([candd, dc], axis=1),
                         jnp.concatenate([candi, iota_n], axis=1)))
    o_ref[...] = (idxs + base).reshape(1, 1, 1, S, K)


def _run_bq(axT, ayT, azT, xpl4, ypl4, zpl4):
    anch_spec = pl.BlockSpec((1, 1, S, 1), lambda t, b, dt: (t, b, 0, 0))
    pts_spec = pl.BlockSpec(
        (1, 1, 1, N), lambda t, b, dt: (jnp.clip(t + dt - 1, 0, T - 1), b, 0, 0))
    return pl.pallas_call(
        _bq_body,
        grid=(T, B, 3),
        in_specs=[anch_spec] * 3 + [pts_spec] * 3,
        out_specs=pl.BlockSpec((1, 1, 1, S, K),
                               lambda t, b, dt: (t, dt, b, 0, 0)),
        out_shape=jax.ShapeDtypeStruct((T, 3, B, S, K), jnp.int32),
    )(axT, ayT, azT, xpl4, ypl4, zpl4)


_NW = 32
_CHUNK = 1024
_IROW = 128
_WROWS = NROWS // _NW
_NCHUNK = _WROWS // _CHUNK


def _sc_gather_body(tab_hbm, idx_hbm, out_hbm, idx_v, rows_v, sem):
    c = lax.axis_index("c")
    s = lax.axis_index("s")
    wid = s * 2 + c

    def chunk(ci, _):
        base = pl.multiple_of(wid * _WROWS + ci * _CHUNK, _CHUNK)
        irow = pl.multiple_of(base // _IROW, _CHUNK // _IROW)
        pltpu.sync_copy(idx_hbm.at[pl.ds(irow, _CHUNK // _IROW)], idx_v)
        handles = []
        for j in range(_CHUNK // _IROW):
            handles.append(pltpu.async_copy(
                tab_hbm.at[idx_v.at[j]],
                rows_v.at[pl.ds(j * _IROW, _IROW)], sem))
        for h in handles:
            h.wait()
        pltpu.sync_copy(rows_v, out_hbm.at[pl.ds(base, _CHUNK)])
        return 0

    lax.fori_loop(0, _NCHUNK, chunk, 0)


def _sc_gather(P_all, idx2d):
    mesh = plsc.VectorSubcoreMesh(core_axis_name="c", subcore_axis_name="s")
    f = functools.partial(
        pl.kernel,
        mesh=mesh,
        compiler_params=pltpu.CompilerParams(use_tc_tiling_on_sc=False),
        out_type=jax.ShapeDtypeStruct((NROWS, MLP0), jnp.float32),
        scratch_types=[
            pltpu.VMEM((_CHUNK // _IROW, _IROW), jnp.int32),
            pltpu.VMEM((_CHUNK, MLP0), jnp.float32),
            pltpu.SemaphoreType.DMA,
        ],
    )(_sc_gather_body)
    return f(P_all, idx2d)


def _mlp_body(g_ref, axr, ayr, azr, wdt_ref, w1b_ref, o_ref):
    dt_id = pl.program_id(2)
    w0 = wdt_ref[0:1, :]
    w1 = wdt_ref[1:2, :]
    w2 = wdt_ref[2:3, :]
    w3 = wdt_ref[3:4, :]
    axv = axr[...].reshape(S, 1)
    ayv = ayr[...].reshape(S, 1)
    azv = azr[...].reshape(S, 1)
    A = axv * w0 + ayv * w1 + azv * w2
    cdt = (dt_id - 1).astype(jnp.float32) * w3
    AB = A - cdt
    g = g_ref[...].reshape(S * K // 8, 8 * MLP0)
    ABr = jnp.broadcast_to(AB.reshape(S, 1, MLP0),
                           (S, K // 8, MLP0)).reshape(S * K // 8, MLP0)
    ABp = jnp.concatenate([ABr] * 8, axis=1)
    r1p = jnp.maximum(g - ABp, 0.0)
    z = jnp.dot(r1p, w1b_ref[...],
                preferred_element_type=jnp.float32,
                precision=lax.Precision.HIGHEST)
    r2 = jnp.maximum(z, 0.0)
    m4 = jnp.max(r2.reshape(S, K // 8, 8 * MLP1), axis=1)
    f = m4[:, 0:MLP1]
    for jg in range(1, 8):
        f = jnp.maximum(f, m4[:, jg * MLP1:(jg + 1) * MLP1])

    @pl.when(dt_id == 0)
    def _():
        o_ref[...] = f.reshape(1, 1, S, MLP1)

    @pl.when(dt_id != 0)
    def _():
        o_ref[...] = o_ref[...] + f.reshape(1, 1, S, MLP1)


def _run_mlp(G5, axT, ayT, azT, WdT, W1blk):
    anch_spec = pl.BlockSpec((1, 1, S, 1), lambda t, b, dt: (t, b, 0, 0))
    return pl.pallas_call(
        _mlp_body,
        grid=(T, B, 3),
        in_specs=[
            pl.BlockSpec((1, 1, 1, S * K // 8, 8 * MLP0),
                         lambda t, b, dt: (t, dt, b, 0, 0)),
            anch_spec, anch_spec, anch_spec,
            pl.BlockSpec((4, MLP0), lambda t, b, dt: (0, 0)),
            pl.BlockSpec((8 * MLP0, 8 * MLP1), lambda t, b, dt: (0, 0)),
        ],
        out_specs=pl.BlockSpec((1, 1, S, MLP1),
                               lambda t, b, dt: (t, b, 0, 0)),
        out_shape=jax.ShapeDtypeStruct((T, B, S, MLP1), jnp.float32),
    )(G5, axT, ayT, azT, WdT, W1blk)


def kernel(points, W_d, W_mlp1):
    xyz_tb = jnp.transpose(points[..., :3], (1, 0, 2, 3))
    xpl = xyz_tb[..., 0].reshape(RT, N)
    ypl = xyz_tb[..., 1].reshape(RT, N)
    zpl = xyz_tb[..., 2].reshape(RT, N)

    ax, ay, az = _run_fps(xpl, ypl, zpl)
    anch = jnp.stack([ax, ay, az], axis=-1).reshape(T, B, S, 3)
    new_xyzs = jnp.transpose(anch, (1, 0, 2, 3))

    axT = ax.reshape(T, B, S, 1)
    ayT = ay.reshape(T, B, S, 1)
    azT = az.reshape(T, B, S, 1)

    xyz8 = jnp.pad(xyz_tb.reshape(RT * N, 3), ((0, 0), (0, 5)))
    wdt8 = jnp.zeros((8, MLP0), jnp.float32).at[:3].set(W_d[:, :3].T)
    P_all = _run_ptab(xyz8, wdt8)

    xpl4 = xpl.reshape(T, B, 1, N)
    ypl4 = ypl.reshape(T, B, 1, N)
    zpl4 = zpl.reshape(T, B, 1, N)
    idxg = _run_bq(axT, ayT, azT, xpl4, ypl4, zpl4)

    idx2d = idxg.reshape(NROWS // _IROW, _IROW)
    G = _sc_gather(P_all, idx2d)
    G5 = G.reshape(T, 3, B, S * K // 8, 8 * MLP0)

    WdT = W_d.T
    W1blk = jnp.kron(jnp.eye(8, dtype=jnp.float32), W_mlp1.T)
    F = _run_mlp(G5, axT, ayT, azT, WdT, W1blk)
    new_features = jnp.transpose(F, (1, 0, 3, 2))
    return new_xyzs, new_features


# speedup vs baseline: 1.8942x; 1.8942x over previous
import functools

import jax
import jax.numpy as jnp
from jax import lax
from jax.experimental import pallas as pl
from jax.experimental.pallas import tpu as pltpu
from jax.experimental.pallas import tpu_sc as plsc

R2 = 0.9 * 0.9
K = 32
N = 2048
S = 512
T = 4
B = 8
RT = T * B
MLP0 = 32
MLP1 = 32
_TJ = (0, 0, 1, 1, 1, 2, 2, 2, 3, 3)
_IJ = (0, 1, 0, 1, 2, 1, 2, 3, 2, 3)
_JTH = ((0, 0, 1, 2, 3, 4), (0, 1, 2, 3, 4, 4))
NJ = 10
JH = 5
HROWS = JH * B * S * K
_NBK = 16
_BW = N // _NBK
_RND = 14
_CW = _NBK * _RND


def _fps_body(x_ref, y_ref, z_ref, ax_ref, ay_ref, az_ref):
    x = x_ref[...]
    y = y_ref[...]
    z = z_ref[...]
    iota_n = lax.broadcasted_iota(jnp.int32, (RT, N), 1)
    iota_s = lax.broadcasted_iota(jnp.int32, (RT, S), 1)

    def step(i, carry):
        dist, far, axc, ayc, azc = carry
        onehot = iota_n == far
        cx = jnp.sum(jnp.where(onehot, x, 0.0), axis=1, keepdims=True)
        cy = jnp.sum(jnp.where(onehot, y, 0.0), axis=1, keepdims=True)
        cz = jnp.sum(jnp.where(onehot, z, 0.0), axis=1, keepdims=True)
        axc = jnp.where(iota_s == i, cx, axc)
        ayc = jnp.where(iota_s == i, cy, ayc)
        azc = jnp.where(iota_s == i, cz, azc)
        dx = x - cx
        dy = y - cy
        dz = z - cz
        d = dx * dx + dy * dy + dz * dz
        dist = jnp.minimum(dist, d)
        m = jnp.max(dist, axis=1, keepdims=True)
        far = jnp.min(jnp.where(dist == m, iota_n, N), axis=1, keepdims=True)
        return dist, far, axc, ayc, azc

    dist0 = jnp.full((RT, N), 1e10, dtype=jnp.float32)
    far0 = jnp.zeros((RT, 1), dtype=jnp.int32)
    acc0 = jnp.zeros((RT, S), dtype=jnp.float32)
    _, _, axc, ayc, azc = lax.fori_loop(
        0, S, step, (dist0, far0, acc0, acc0, acc0))
    ax_ref[...] = axc
    ay_ref[...] = ayc
    az_ref[...] = azc


def _run_fps(xpl, ypl, zpl):
    out = jax.ShapeDtypeStruct((RT, S), jnp.float32)
    return pl.pallas_call(
        _fps_body,
        out_shape=(out, out, out),
    )(xpl, ypl, zpl)


def _ptab_body(x_ref, w_ref, o_ref):
    o_ref[...] = jnp.dot(x_ref[...], w_ref[...],
                         preferred_element_type=jnp.float32,
                         precision=lax.Precision.HIGHEST)


def _run_ptab(xyz8, wdt8):
    grid = 8
    rows = RT * N // grid
    return pl.pallas_call(
        _ptab_body,
        grid=(grid,),
        in_specs=[
            pl.BlockSpec((rows, 8), lambda g: (g, 0)),
            pl.BlockSpec((8, MLP0), lambda g: (0, 0)),
        ],
        out_specs=pl.BlockSpec((rows, MLP0), lambda g: (g, 0)),
        out_shape=jax.ShapeDtypeStruct((RT * N, MLP0), jnp.float32),
    )(xyz8, wdt8)


def _bq_body(jlo, base_ref, axr, ayr, azr, xr, yr, zr, o_ref):
    base = base_ref[jlo + pl.program_id(0), pl.program_id(1)]
    ax = axr[...].reshape(S, 1)
    ay = ayr[...].reshape(S, 1)
    az = azr[...].reshape(S, 1)
    xn = xr[...].reshape(1, N)
    yn = yr[...].reshape(1, N)
    zn = zr[...].reshape(1, N)
    a8 = jnp.concatenate(
        [ax, ay, az, jnp.zeros((S, 5), jnp.float32)], axis=1)
    x8 = jnp.concatenate(
        [xn, yn, zn, jnp.zeros((5, N), jnp.float32)], axis=0)
    dot = jnp.dot(a8, x8, preferred_element_type=jnp.float32)
    sa = ax * ax + ay * ay + az * az
    sx = xn * xn + yn * yn + zn * zn
    dmat = (-2.0 * dot + sa) + sx

    iota_n = lax.broadcasted_iota(jnp.int32, (S, N), 1)
    iota_k = lax.broadcasted_iota(jnp.int32, (S, K), 1)

    def step(k, carry):
        dc, idxs, sel0 = carry
        m = jnp.min(dc, axis=1, keepdims=True)
        sel = jnp.min(jnp.where(dc == m, iota_n, N), axis=1, keepdims=True)
        sel0 = jnp.where(k == 0, sel, sel0)
        eff = jnp.where(m > R2, sel0, sel)
        idxs = jnp.where(iota_k == k, eff, idxs)
        dc = jnp.where(iota_n == sel, jnp.inf, dc)
        return dc, idxs, sel0

    idxs0 = jnp.zeros((S, K), dtype=jnp.int32)
    sel00 = jnp.zeros((S, 1), dtype=jnp.int32)
    _, idxs, _ = lax.fori_loop(0, K, step, (dmat, idxs0, sel00))
    o_ref[...] = (idxs + base).reshape(1, 1, S, K)


def _run_bq(base_arr, axT, ayT, azT, xpl4, ypl4, zpl4, jlo):
    anch_spec = pl.BlockSpec(
        (1, 1, S, 1),
        lambda j, b: ((jlo + j + 1) // 3, b, 0, 0))
    pts_spec = pl.BlockSpec(
        (1, 1, 1, N),
        lambda j, b: ((jlo + j) - 2 * ((jlo + j + 1) // 3), b, 0, 0))
    return pl.pallas_call(
        functools.partial(_bq_body, jlo),
        grid=(JH, B),
        in_specs=[pl.BlockSpec(memory_space=pltpu.SMEM)]
        + [anch_spec] * 3 + [pts_spec] * 3,
        out_specs=pl.BlockSpec((1, 1, S, K), lambda j, b: (j, b, 0, 0)),
        out_shape=jax.ShapeDtypeStruct((JH, B, S, K), jnp.int32),
    )(base_arr, axT, ayT, azT, xpl4, ypl4, zpl4)


_NW = 32
_CHUNK = 1024
_IROW = 128
_WROWS = HROWS // _NW
_NCHUNK = _WROWS // _CHUNK


def _sc_gather_body(tab_hbm, idx_hbm, out_hbm, idx_v, rows_v, sem):
    c = lax.axis_index("c")
    s = lax.axis_index("s")
    wid = s * 2 + c

    def chunk(ci, _):
        base = pl.multiple_of(wid * _WROWS + ci * _CHUNK, _CHUNK)
        irow = pl.multiple_of(base // _IROW, _CHUNK // _IROW)
        pltpu.sync_copy(idx_hbm.at[pl.ds(irow, _CHUNK // _IROW)], idx_v)
        handles = []
        for j in range(_CHUNK // _IROW):
            handles.append(pltpu.async_copy(
                tab_hbm.at[idx_v.at[j]],
                rows_v.at[pl.ds(j * _IROW, _IROW)], sem))
        for h in handles:
            h.wait()
        pltpu.sync_copy(rows_v, out_hbm.at[pl.ds(base, _CHUNK)])
        return 0

    lax.fori_loop(0, _NCHUNK, chunk, 0)


def _sc_gather(P_all, idx2d):
    mesh = plsc.VectorSubcoreMesh(core_axis_name="c", subcore_axis_name="s")
    f = functools.partial(
        pl.kernel,
        mesh=mesh,
        compiler_params=pltpu.CompilerParams(use_tc_tiling_on_sc=False),
        out_type=jax.ShapeDtypeStruct((HROWS, MLP0), jnp.float32),
        scratch_types=[
            pltpu.VMEM((_CHUNK // _IROW, _IROW), jnp.int32),
            pltpu.VMEM((_CHUNK, MLP0), jnp.float32),
            pltpu.SemaphoreType.DMA,
        ],
    )(_sc_gather_body)
    return f(P_all, idx2d)


def _mlp_body(g_ref, axr, ayr, azr, wdt_ref, w1b_ref, o_ref):
    dt_id = pl.program_id(2)
    w0 = wdt_ref[0:1, :]
    w1 = wdt_ref[1:2, :]
    w2 = wdt_ref[2:3, :]
    w3 = wdt_ref[3:4, :]
    axv = axr[...].reshape(S, 1)
    ayv = ayr[...].reshape(S, 1)
    azv = azr[...].reshape(S, 1)
    A = axv * w0 + ayv * w1 + azv * w2
    cdt = (dt_id - 1).astype(jnp.float32) * w3
    AB = A - cdt
    g = g_ref[...].reshape(S * K // 8, 8 * MLP0)
    ABr = jnp.broadcast_to(AB.reshape(S, 1, MLP0),
                           (S, K // 8, MLP0)).reshape(S * K // 8, MLP0)
    ABp = jnp.concatenate([ABr] * 8, axis=1)
    r1p = jnp.maximum(g - ABp, 0.0)
    z = jnp.dot(r1p, w1b_ref[...],
                preferred_element_type=jnp.float32,
                precision=lax.Precision.HIGHEST)
    r2 = jnp.maximum(z, 0.0)
    m4 = jnp.max(r2.reshape(S, K // 8, 8 * MLP1), axis=1)
    f = m4[:, 0:MLP1]
    for jg in range(1, 8):
        f = jnp.maximum(f, m4[:, jg * MLP1:(jg + 1) * MLP1])

    @pl.when(dt_id == 0)
    def _():
        o_ref[...] = f.reshape(1, 1, S, MLP1)

    @pl.when(dt_id != 0)
    def _():
        o_ref[...] = o_ref[...] + f.reshape(1, 1, S, MLP1)


def _run_mlp(G4, axT, ayT, azT, WdT, W1blk, h):
    jth = _JTH[h]
    anch_spec = pl.BlockSpec((1, 1, S, 1),
                             lambda t, b, dt: (h * 2 + t, b, 0, 0))
    return pl.pallas_call(
        _mlp_body,
        grid=(2, B, 3),
        in_specs=[
            pl.BlockSpec((1, 1, S * K // 8, 8 * MLP0),
                         lambda t, b, dt: (
                             jnp.maximum(3 * (h * 2 + t) - 1, 0)
                             + jnp.clip((h * 2 + t) + dt - 1, 0, T - 1)
                             - jnp.maximum((h * 2 + t) - 1, 0) - JH * h,
                             b, 0, 0)),
            anch_spec, anch_spec, anch_spec,
            pl.BlockSpec((4, MLP0), lambda t, b, dt: (0, 0)),
            pl.BlockSpec((8 * MLP0, 8 * MLP1), lambda t, b, dt: (0, 0)),
        ],
        out_specs=pl.BlockSpec((1, 1, S, MLP1),
                               lambda t, b, dt: (t, b, 0, 0)),
        out_shape=jax.ShapeDtypeStruct((2, B, S, MLP1), jnp.float32),
    )(G4, axT, ayT, azT, WdT, W1blk)


def kernel(points, W_d, W_mlp1):
    xyz_tb = jnp.transpose(points[..., :3], (1, 0, 2, 3))
    xpl = xyz_tb[..., 0].reshape(RT, N)
    ypl = xyz_tb[..., 1].reshape(RT, N)
    zpl = xyz_tb[..., 2].reshape(RT, N)

    ax, ay, az = _run_fps(xpl, ypl, zpl)
    anch = jnp.stack([ax, ay, az], axis=-1).reshape(T, B, S, 3)
    new_xyzs = jnp.transpose(anch, (1, 0, 2, 3))

    axT = ax.reshape(T, B, S, 1)
    ayT = ay.reshape(T, B, S, 1)
    azT = az.reshape(T, B, S, 1)

    xyz8 = jnp.pad(xyz_tb.reshape(RT * N, 3), ((0, 0), (0, 5)))
    wdt8 = jnp.zeros((8, MLP0), jnp.float32).at[:3].set(W_d[:, :3].T)
    P_all = _run_ptab(xyz8, wdt8)

    xpl4 = xpl.reshape(T, B, 1, N)
    ypl4 = ypl.reshape(T, B, 1, N)
    zpl4 = zpl.reshape(T, B, 1, N)

    base_arr = ((jnp.array(_IJ, jnp.int32) * B)[:, None]
                + jnp.arange(B, dtype=jnp.int32)[None, :]) * N
    idx1 = _run_bq(base_arr, axT, ayT, azT, xpl4, ypl4, zpl4, 0)
    idx2 = _run_bq(base_arr, axT, ayT, azT, xpl4, ypl4, zpl4, JH)

    G1 = _sc_gather(P_all, idx1.reshape(HROWS // _IROW, _IROW))
    G2 = _sc_gather(P_all, idx2.reshape(HROWS // _IROW, _IROW))

    WdT = W_d.T
    W1blk = jnp.kron(jnp.eye(8, dtype=jnp.float32), W_mlp1.T)
    F1 = _run_mlp(G1.reshape(JH, B, S * K // 8, 8 * MLP0),
                  axT, ayT, azT, WdT, W1blk, 0)
    F2 = _run_mlp(G2.reshape(JH, B, S * K // 8, 8 * MLP0),
                  axT, ayT, azT, WdT, W1blk, 1)
    F = jnp.concatenate([F1, F2], axis=0)
    new_features = jnp.transpose(F, (1, 0, 3, 2))
    return new_xyzs, new_features
